# Initial kernel scaffold; baseline (speedup 1.0000x reference)
import jax
import jax.numpy as jnp
from jax.experimental import pallas as pl


def kernel(x, edge_index, W1_l, b1, W1_r, W2_l, b2, W2_r):
    raise NotImplementedError("write your pallas kernel here")


# speedup vs baseline: 11.8940x; 11.8940x over previous
import functools

import jax
import jax.numpy as jnp
from jax import lax
from jax.experimental import pallas as pl
from jax.experimental.pallas import tpu as pltpu
from jax.experimental.pallas import tpu_sc as plsc

N = 10000
D = 128
E = 320000

NC = 2
NS = 16
NW = NC * NS
K = 64
E_PAD = 327680
CHUNKS = E_PAD // (NW * K)
IB = 32
GROUPS = CHUNKS // IB
ROWS2D = E_PAD // K
N_PAD = 10240
STRIPE = N_PAD // NS

_f32 = jnp.float32


def _seg_body(table, srci, dsti, zfull, out,
              acc, idx_s, idx_d, r0, r1, r2,
              sg0, sg1, sg2, ss0, ss1, ss2):
    c = lax.axis_index("c")
    s = lax.axis_index("s")
    wid = s * NC + c
    base = s * STRIPE

    pltpu.sync_copy(zfull.at[pl.ds(base, STRIPE)], acc.at[pl.ds(base, STRIPE)])
    plsc.subcore_barrier()

    bufs = (r0, r1, r2)
    gsem = (sg0, sg1, sg2)
    ssem = (ss0, ss1, ss2)

    def g_start(j):
        return pltpu.async_copy(table.at[idx_s.at[j]], bufs[j % 3],
                                gsem[j % 3])

    def s_start(j):
        return pltpu.async_copy(bufs[j % 3], acc.at[idx_d.at[j]],
                                ssem[j % 3], add=True)

    @pl.loop(0, GROUPS)
    def _(g):
        grow = wid * CHUNKS + g * IB
        pltpu.sync_copy(srci.at[pl.ds(grow, IB)], idx_s)
        pltpu.sync_copy(dsti.at[pl.ds(grow, IB)], idx_d)

        d_g = [None] * IB
        d_s = [None] * IB
        for j in range(3):
            d_g[j] = g_start(j)
        for j in range(IB):
            d_g[j].wait()
            d_s[j] = s_start(j)
            if 1 <= j < IB - 2:
                d_s[j - 1].wait()
                d_g[j + 2] = g_start(j + 2)
        for j in (IB - 3, IB - 2, IB - 1):
            d_s[j].wait()

    plsc.subcore_barrier()
    pltpu.sync_copy(acc.at[pl.ds(base, STRIPE)],
                    out.at[c, pl.ds(base, STRIPE)])


def _cnt_body(dsti, zcnt, cnt_out, cnt_sh, idx_d, aux, sg, ss):
    c = lax.axis_index("c")
    s = lax.axis_index("s")
    wid = s * NC + c
    base = s * STRIPE

    pltpu.sync_copy(zcnt.at[pl.ds(base, STRIPE)],
                    cnt_sh.at[pl.ds(base, STRIPE)])

    @pl.loop(0, K)
    def _(r):
        aux[r, pl.ds(0, 16)] = jnp.ones((16,), _f32)

    plsc.subcore_barrier()

    @pl.loop(0, GROUPS)
    def _(g):
        grow = wid * CHUNKS + g * IB
        pltpu.sync_copy(dsti.at[pl.ds(grow, IB)], idx_d)
        d = [pltpu.async_copy(aux, cnt_sh.at[idx_d.at[j]], ss, add=True)
             for j in range(IB)]
        for dd in d:
            dd.wait()

    plsc.subcore_barrier()
    pltpu.sync_copy(cnt_sh.at[pl.ds(base, STRIPE)],
                    cnt_out.at[c, pl.ds(base, STRIPE)])


_SC_MESH = plsc.VectorSubcoreMesh(core_axis_name="c", subcore_axis_name="s",
                                  num_cores=NC, num_subcores=NS)
_SC_PARAMS = pltpu.CompilerParams(use_tc_tiling_on_sc=False)

_seg_sum = pl.kernel(
    _seg_body,
    out_type=jax.ShapeDtypeStruct((NC, N_PAD, D), _f32),
    mesh=_SC_MESH,
    scratch_types=[
        pltpu.VMEM_SHARED((N_PAD, D), _f32),
        pltpu.VMEM((IB, K), jnp.int32),
        pltpu.VMEM((IB, K), jnp.int32),
        pltpu.VMEM((K, D), _f32),
        pltpu.VMEM((K, D), _f32),
        pltpu.VMEM((K, D), _f32),
        pltpu.SemaphoreType.DMA,
        pltpu.SemaphoreType.DMA,
        pltpu.SemaphoreType.DMA,
        pltpu.SemaphoreType.DMA,
        pltpu.SemaphoreType.DMA,
        pltpu.SemaphoreType.DMA,
    ],
    compiler_params=_SC_PARAMS,
)

_cnt_sum = pl.kernel(
    _cnt_body,
    out_type=jax.ShapeDtypeStruct((NC, N_PAD, 16), _f32),
    mesh=_SC_MESH,
    scratch_types=[
        pltpu.VMEM_SHARED((N_PAD, 16), _f32),
        pltpu.VMEM((IB, K), jnp.int32),
        pltpu.VMEM((K, 16), _f32),
        pltpu.SemaphoreType.DMA,
        pltpu.SemaphoreType.DMA,
    ],
    compiler_params=_SC_PARAMS,
)


_R = 1000


def _tck1_body(x_ref, wl_ref, wr_ref, b_ref, xl_ref, xr_ref):
    xb = x_ref[...]
    xl_ref[...] = jnp.dot(xb, wl_ref[...], preferred_element_type=_f32)
    xr_ref[...] = jnp.dot(xb, wr_ref[...], preferred_element_type=_f32) + b_ref[...]


def _tck1(x, W_l, W_r, b):
    return pl.pallas_call(
        _tck1_body,
        grid=(N // _R,),
        in_specs=[
            pl.BlockSpec((_R, D), lambda i: (i, 0)),
            pl.BlockSpec((D, D), lambda i: (0, 0)),
            pl.BlockSpec((D, D), lambda i: (0, 0)),
            pl.BlockSpec((1, D), lambda i: (0, 0)),
        ],
        out_specs=[
            pl.BlockSpec((_R, D), lambda i: (i, 0)),
            pl.BlockSpec((_R, D), lambda i: (i, 0)),
        ],
        out_shape=[
            jax.ShapeDtypeStruct((N, D), _f32),
            jax.ShapeDtypeStruct((N, D), _f32),
        ],
    )(x, W_l, W_r, b)


def _tck2_body(agg_ref, cnt_ref, xr_ref, wl_ref, wr_ref, b_ref,
               hl_ref, hr_ref, inv_ref):
    cnt = cnt_ref[0, :, 0:1] + cnt_ref[1, :, 0:1]
    inv = 1.0 / jnp.maximum(cnt, 1.0)
    h = jnp.maximum((agg_ref[0] + agg_ref[1]) * inv + xr_ref[...], 0.0)
    hl_ref[...] = jnp.dot(h, wl_ref[...], preferred_element_type=_f32)
    hr_ref[...] = jnp.dot(h, wr_ref[...], preferred_element_type=_f32) + b_ref[...]
    inv_ref[...] = jnp.broadcast_to(inv, inv_ref.shape)


def _tck2(aggp, cntp, xr1b, W_l, W_r, b):
    return pl.pallas_call(
        _tck2_body,
        grid=(N // _R,),
        in_specs=[
            pl.BlockSpec((NC, _R, D), lambda i: (0, i, 0)),
            pl.BlockSpec((NC, _R, 16), lambda i: (0, i, 0)),
            pl.BlockSpec((_R, D), lambda i: (i, 0)),
            pl.BlockSpec((D, D), lambda i: (0, 0)),
            pl.BlockSpec((D, D), lambda i: (0, 0)),
            pl.BlockSpec((1, D), lambda i: (0, 0)),
        ],
        out_specs=[
            pl.BlockSpec((_R, D), lambda i: (i, 0)),
            pl.BlockSpec((_R, D), lambda i: (i, 0)),
            pl.BlockSpec((_R, 16), lambda i: (i, 0)),
        ],
        out_shape=[
            jax.ShapeDtypeStruct((N, D), _f32),
            jax.ShapeDtypeStruct((N, D), _f32),
            jax.ShapeDtypeStruct((N, 16), _f32),
        ],
    )(aggp, cntp, xr1b, W_l, W_r, b)


def _tck3_body(agg_ref, inv_ref, hr_ref, out_ref):
    out_ref[...] = (agg_ref[0] + agg_ref[1]) * inv_ref[:, 0:1] + hr_ref[...]


def _tck3(aggp, invc, hr2b):
    return pl.pallas_call(
        _tck3_body,
        grid=(N // _R,),
        in_specs=[
            pl.BlockSpec((NC, _R, D), lambda i: (0, i, 0)),
            pl.BlockSpec((_R, 16), lambda i: (i, 0)),
            pl.BlockSpec((_R, D), lambda i: (i, 0)),
        ],
        out_specs=pl.BlockSpec((_R, D), lambda i: (i, 0)),
        out_shape=jax.ShapeDtypeStruct((N, D), _f32),
    )(aggp, invc, hr2b)


def kernel(x, edge_index, W1_l, b1, W1_r, W2_l, b2, W2_r):
    ei = edge_index.astype(jnp.int32)
    src, dst = ei[0], ei[1]
    pad = E_PAD - E
    pad_i = jnp.arange(pad, dtype=jnp.int32)
    src = jnp.concatenate([src, pad_i % N]).reshape(ROWS2D, K)
    dst = jnp.concatenate([dst, N + pad_i % (N_PAD - N)]).reshape(ROWS2D, K)

    b1r = b1.reshape(1, D)
    b2r = b2.reshape(1, D)
    zfull = jnp.zeros((N_PAD, D), _f32)
    zcnt = jnp.zeros((N_PAD, 16), _f32)

    cntp = _cnt_sum(dst, zcnt)

    xl1, xr1b = _tck1(x, W1_l, W1_r, b1r)
    agg1p = _seg_sum(xl1, src, dst, zfull)
    hl2, hr2b, invc = _tck2(agg1p, cntp, xr1b, W2_l, W2_r, b2r)
    agg2p = _seg_sum(hl2, src, dst, zfull)
    return _tck3(agg2p, invc, hr2b)
